# baseline (speedup 1.0000x reference)
import functools

import numpy as np
import jax
import jax.numpy as jnp
from jax import lax
from jax.experimental import pallas as pl
from jax.experimental.pallas import tpu as pltpu
from jax.experimental.pallas import tpu_sc as plsc

_NU = 768
_SCALE = float(np.sqrt(float(_NU)))
_L = 16
_GROUP = 32

_TWO_PI = 2.0 * np.pi
_CW1 = 6.28125
_CW2 = float(np.float32(_TWO_PI - _CW1))
_CW3 = float(np.float64(_TWO_PI) - _CW1 - np.float32(_TWO_PI - _CW1))
_INV_2PI = float(1.0 / _TWO_PI)


@functools.cache
def _poly_coeffs():
    r = np.linspace(1e-9, np.pi, 4001)
    r2 = r * r
    vs = np.stack([r2**k for k in range(6)], axis=1)
    ps = np.linalg.lstsq(vs, np.sin(r) / r, rcond=None)[0]
    vc = np.stack([r2**k for k in range(7)], axis=1)
    pc = np.linalg.lstsq(vc, np.cos(r), rcond=None)[0]
    return [float(x) for x in ps], [float(x) for x in pc]


def _sincos(x):
    ps, pc = _poly_coeffs()
    n = (x * jnp.float32(_INV_2PI) + jnp.float32(0.5)).astype(jnp.int32)
    nf = n.astype(jnp.float32)
    r = ((x - nf * jnp.float32(_CW1)) - nf * jnp.float32(_CW2)) - nf * jnp.float32(_CW3)
    r2 = r * r
    s = jnp.float32(ps[-1])
    for a in ps[-2::-1]:
        s = s * r2 + jnp.float32(a)
    s = s * r
    c = jnp.float32(pc[-1])
    for a in pc[-2::-1]:
        c = c * r2 + jnp.float32(a)
    return s, c


@functools.cache
def _consts():
    i = np.arange(_NU, dtype=np.float64)
    inv = 1.0 / np.power(10000.0, 2.0 * i / _NU)
    return (jnp.asarray(inv, dtype=jnp.float32),
            jnp.asarray(np.cos(inv), dtype=jnp.float32),
            jnp.asarray(np.sin(inv), dtype=jnp.float32))


@functools.cache
def _make_posenc(N: int, T: int):
    info = plsc.get_sparse_core_info()
    nw = info.num_cores * info.num_subcores
    rpw = T // nw
    ngroups = rpw // _GROUP
    nchunk = _NU // _L
    gsz = _GROUP * _NU
    mesh = plsc.VectorSubcoreMesh(core_axis_name="c", subcore_axis_name="s")

    @functools.partial(
        pl.kernel,
        mesh=mesh,
        out_type=jax.ShapeDtypeStruct((N, T * _NU), jnp.float32),
        scratch_types=[
            pltpu.VMEM((_NU,), jnp.float32),
            pltpu.VMEM((_NU,), jnp.float32),
            pltpu.VMEM((_NU,), jnp.float32),
            pltpu.VMEM((_NU,), jnp.float32),
            pltpu.VMEM((_NU,), jnp.float32),
            pltpu.VMEM((gsz,), jnp.float32),
            pltpu.VMEM((gsz,), jnp.float32),
            pltpu.SemaphoreType.DMA,
            pltpu.SemaphoreType.DMA,
        ],
    )
    def posenc(inv_hbm, c1_hbm, s1_hbm, out_hbm,
               inv_v, c1_v, s1_v, av, bv, bufa, bufb, sema, semb):
        wid = lax.axis_index("s") * info.num_cores + lax.axis_index("c")
        base = wid * rpw
        pltpu.sync_copy(inv_hbm, inv_v)
        pltpu.sync_copy(c1_hbm, c1_v)
        pltpu.sync_copy(s1_hbm, s1_v)

        m = jnp.float32(1.0) - (lax.broadcasted_iota(jnp.int32, (_L,), 0)
                                & 1).astype(jnp.float32)
        posf = base.astype(jnp.float32)

        bufs = (bufa, bufb)
        sems = (sema, semb)
        pending = []

        for g in range(ngroups):
            buf = bufs[g % 2]
            sem = sems[g % 2]
            if g >= 2:
                for h in pending.pop(0):
                    h.wait()

            first = g == 0

            def chunk_body(j, _, first=first, buf=buf):
                c1 = c1_v[pl.ds(j * _L, _L)]
                s1 = s1_v[pl.ds(j * _L, _L)]
                if first:
                    x = inv_v[pl.ds(j * _L, _L)] * posf
                    s, c = _sincos(x)
                    s = s * jnp.float32(_SCALE)
                    c = c * jnp.float32(_SCALE)
                    state = (c + (s - c) * m, -s + (c + s) * m)
                else:
                    state = (av[pl.ds(j * _L, _L)], bv[pl.ds(j * _L, _L)])

                def row_body(p, ab, j=j, buf=buf):
                    a, b = ab
                    buf[pl.ds(p * _NU + j * _L, _L)] = a
                    return (a * c1 + b * s1, b * c1 - a * s1)

                a, b = lax.fori_loop(0, _GROUP, row_body, state)
                av[pl.ds(j * _L, _L)] = a
                bv[pl.ds(j * _L, _L)] = b
                return 0

            lax.fori_loop(0, nchunk, chunk_body, 0)

            if g == 0:
                @pl.when(wid == 0)
                def _zero_row():
                    for j in range(nchunk):
                        buf[pl.ds(j * _L, _L)] = jnp.zeros((_L,), jnp.float32)

            handles = [
                pltpu.async_copy(
                    buf, out_hbm.at[n, pl.ds((base + g * _GROUP) * _NU, gsz)],
                    sem)
                for n in range(N)
            ]
            pending.append(handles)

        for hs in pending:
            for h in hs:
                h.wait()

    return posenc


def kernel(inputs):
    N, T = inputs.shape[0], inputs.shape[1]
    inv, c1, s1 = _consts()
    flat = _make_posenc(N, T)(inv, c1, s1)
    return flat.reshape(N, T, _NU)


# speedup vs baseline: 1.1350x; 1.1350x over previous
import functools

import numpy as np
import jax
import jax.numpy as jnp
from jax import lax
from jax.experimental import pallas as pl
from jax.experimental.pallas import tpu as pltpu
from jax.experimental.pallas import tpu_sc as plsc

_NU = 768
_SCALE = float(np.sqrt(float(_NU)))
_L = 16
_GROUP = 32
_CH = 4

_TWO_PI = 2.0 * np.pi
_CW1 = 6.28125
_CW2 = float(np.float32(_TWO_PI - _CW1))
_CW3 = float(np.float64(_TWO_PI) - _CW1 - np.float32(_TWO_PI - _CW1))
_INV_2PI = float(1.0 / _TWO_PI)


@functools.cache
def _poly_coeffs():
    r = np.linspace(1e-9, np.pi, 4001)
    r2 = r * r
    vs = np.stack([r2**k for k in range(6)], axis=1)
    ps = np.linalg.lstsq(vs, np.sin(r) / r, rcond=None)[0]
    vc = np.stack([r2**k for k in range(7)], axis=1)
    pc = np.linalg.lstsq(vc, np.cos(r), rcond=None)[0]
    return [float(x) for x in ps], [float(x) for x in pc]


def _sincos(x):
    ps, pc = _poly_coeffs()
    n = (x * jnp.float32(_INV_2PI) + jnp.float32(0.5)).astype(jnp.int32)
    nf = n.astype(jnp.float32)
    r = ((x - nf * jnp.float32(_CW1)) - nf * jnp.float32(_CW2)) - nf * jnp.float32(_CW3)
    r2 = r * r
    s = jnp.float32(ps[-1])
    for a in ps[-2::-1]:
        s = s * r2 + jnp.float32(a)
    s = s * r
    c = jnp.float32(pc[-1])
    for a in pc[-2::-1]:
        c = c * r2 + jnp.float32(a)
    return s, c


@functools.cache
def _consts():
    i = np.arange(_NU, dtype=np.float64)
    inv = 1.0 / np.power(10000.0, 2.0 * i / _NU)
    return (jnp.asarray(inv, dtype=jnp.float32),
            jnp.asarray(np.cos(inv), dtype=jnp.float32),
            jnp.asarray(np.sin(inv), dtype=jnp.float32))


@functools.cache
def _make_posenc(N: int, T: int):
    info = plsc.get_sparse_core_info()
    nw = info.num_cores * info.num_subcores
    rpw = T // nw
    ngroups = rpw // _GROUP
    nchunk = _NU // _L
    gsz = _GROUP * _NU
    mesh = plsc.VectorSubcoreMesh(core_axis_name="c", subcore_axis_name="s")

    @functools.partial(
        pl.kernel,
        mesh=mesh,
        out_type=jax.ShapeDtypeStruct((N, T * _NU), jnp.float32),
        scratch_types=[
            pltpu.VMEM((_NU,), jnp.float32),
            pltpu.VMEM((_NU,), jnp.float32),
            pltpu.VMEM((_NU,), jnp.float32),
            pltpu.VMEM((_NU,), jnp.float32),
            pltpu.VMEM((_NU,), jnp.float32),
            pltpu.VMEM((gsz,), jnp.float32),
            pltpu.VMEM((gsz,), jnp.float32),
            pltpu.SemaphoreType.DMA,
            pltpu.SemaphoreType.DMA,
        ],
    )
    def posenc(inv_hbm, c1_hbm, s1_hbm, out_hbm,
               inv_v, c1_v, s1_v, av, bv, bufa, bufb, sema, semb):
        wid = lax.axis_index("s") * info.num_cores + lax.axis_index("c")
        base = wid * rpw
        pltpu.sync_copy(inv_hbm, inv_v)
        pltpu.sync_copy(c1_hbm, c1_v)
        pltpu.sync_copy(s1_hbm, s1_v)

        m = jnp.float32(1.0) - (lax.broadcasted_iota(jnp.int32, (_L,), 0)
                                & 1).astype(jnp.float32)
        posf = base.astype(jnp.float32)

        bufs = (bufa, bufb)
        sems = (sema, semb)
        pending = []

        for g in range(ngroups):
            buf = bufs[g % 2]
            sem = sems[g % 2]
            if g >= 2:
                for h in pending.pop(0):
                    h.wait()

            first = g == 0

            def chunk_body(jj, _, first=first, buf=buf):
                jb = jj * (_CH * _L)
                c1s, s1s, a, b = [], [], [], []
                for t in range(_CH):
                    off = jb + t * _L
                    c1s.append(c1_v[pl.ds(off, _L)])
                    s1s.append(s1_v[pl.ds(off, _L)])
                    if first:
                        x = inv_v[pl.ds(off, _L)] * posf
                        s, c = _sincos(x)
                        s = s * jnp.float32(_SCALE)
                        c = c * jnp.float32(_SCALE)
                        a.append(c + (s - c) * m)
                        b.append(-s + (c + s) * m)
                    else:
                        a.append(av[pl.ds(off, _L)])
                        b.append(bv[pl.ds(off, _L)])
                for p in range(_GROUP):
                    for t in range(_CH):
                        buf[pl.ds(p * _NU + jb + t * _L, _L)] = a[t]
                        a[t], b[t] = (a[t] * c1s[t] + b[t] * s1s[t],
                                      b[t] * c1s[t] - a[t] * s1s[t])
                for t in range(_CH):
                    off = jb + t * _L
                    av[pl.ds(off, _L)] = a[t]
                    bv[pl.ds(off, _L)] = b[t]
                return 0

            lax.fori_loop(0, nchunk // _CH, chunk_body, 0)

            if g == 0:
                @pl.when(wid == 0)
                def _zero_row():
                    for j in range(nchunk):
                        buf[pl.ds(j * _L, _L)] = jnp.zeros((_L,), jnp.float32)

            handles = [
                pltpu.async_copy(
                    buf, out_hbm.at[n, pl.ds((base + g * _GROUP) * _NU, gsz)],
                    sem)
                for n in range(N)
            ]
            pending.append(handles)

        for hs in pending:
            for h in hs:
                h.wait()

    return posenc


def kernel(inputs):
    N, T = inputs.shape[0], inputs.shape[1]
    inv, c1, s1 = _consts()
    flat = _make_posenc(N, T)(inv, c1, s1)
    return flat.reshape(N, T, _NU)


# speedup vs baseline: 2.1841x; 1.9243x over previous
import functools

import numpy as np
import jax
import jax.numpy as jnp
from jax import lax
from jax.experimental import pallas as pl
from jax.experimental.pallas import tpu as pltpu
from jax.experimental.pallas import tpu_sc as plsc

_NU = 768
_SCALE = float(np.sqrt(float(_NU)))
_L = 16
_GROUP = 32
_CH = 4

_TWO_PI = 2.0 * np.pi
_CW1 = 6.28125
_CW2 = float(np.float32(_TWO_PI - _CW1))
_CW3 = float(np.float64(_TWO_PI) - _CW1 - np.float32(_TWO_PI - _CW1))
_INV_2PI = float(1.0 / _TWO_PI)


@functools.cache
def _poly_coeffs():
    r = np.linspace(1e-9, np.pi, 4001)
    r2 = r * r
    vs = np.stack([r2**k for k in range(6)], axis=1)
    ps = np.linalg.lstsq(vs, np.sin(r) / r, rcond=None)[0]
    vc = np.stack([r2**k for k in range(7)], axis=1)
    pc = np.linalg.lstsq(vc, np.cos(r), rcond=None)[0]
    return [float(x) for x in ps], [float(x) for x in pc]


def _sincos(x):
    ps, pc = _poly_coeffs()
    n = (x * jnp.float32(_INV_2PI) + jnp.float32(0.5)).astype(jnp.int32)
    nf = n.astype(jnp.float32)
    r = ((x - nf * jnp.float32(_CW1)) - nf * jnp.float32(_CW2)) - nf * jnp.float32(_CW3)
    r2 = r * r
    s = jnp.float32(ps[-1])
    for a in ps[-2::-1]:
        s = s * r2 + jnp.float32(a)
    s = s * r
    c = jnp.float32(pc[-1])
    for a in pc[-2::-1]:
        c = c * r2 + jnp.float32(a)
    return s, c


@functools.cache
def _consts():
    i = np.arange(_NU, dtype=np.float64)
    inv = 1.0 / np.power(10000.0, 2.0 * i / _NU)
    return (jnp.asarray(inv, dtype=jnp.float32),
            jnp.asarray(np.cos(inv), dtype=jnp.float32),
            jnp.asarray(np.sin(inv), dtype=jnp.float32))


@functools.cache
def _make_posenc(N: int, T: int):
    info = plsc.get_sparse_core_info()
    nw = info.num_cores * info.num_subcores
    rpw = T // nw
    ngroups = rpw // _GROUP
    nchunk = _NU // _L
    gsz = _GROUP * _NU
    mesh = plsc.VectorSubcoreMesh(core_axis_name="c", subcore_axis_name="s")

    @functools.partial(
        pl.kernel,
        mesh=mesh,
        out_type=jax.ShapeDtypeStruct((N, T, _NU), jnp.float32),
        scratch_types=[
            pltpu.VMEM((_NU,), jnp.float32),
            pltpu.VMEM((_NU,), jnp.float32),
            pltpu.VMEM((_NU,), jnp.float32),
            pltpu.VMEM((_NU,), jnp.float32),
            pltpu.VMEM((_NU,), jnp.float32),
            pltpu.VMEM((_GROUP, _NU), jnp.float32),
            pltpu.VMEM((_GROUP, _NU), jnp.float32),
            pltpu.SemaphoreType.DMA,
            pltpu.SemaphoreType.DMA,
        ],
    )
    def posenc(inv_hbm, c1_hbm, s1_hbm, out_hbm,
               inv_v, c1_v, s1_v, av, bv, bufa, bufb, sema, semb):
        wid = lax.axis_index("s") * info.num_cores + lax.axis_index("c")
        base = wid * rpw
        pltpu.sync_copy(inv_hbm, inv_v)
        pltpu.sync_copy(c1_hbm, c1_v)
        pltpu.sync_copy(s1_hbm, s1_v)

        m = jnp.float32(1.0) - (lax.broadcasted_iota(jnp.int32, (_L,), 0)
                                & 1).astype(jnp.float32)
        posf = base.astype(jnp.float32)

        bufs = (bufa, bufb)
        sems = (sema, semb)
        pending = []

        for g in range(ngroups):
            buf = bufs[g % 2]
            sem = sems[g % 2]
            if g >= 2:
                for h in pending.pop(0):
                    h.wait()

            first = g == 0

            def chunk_body(jj, _, first=first, buf=buf):
                jb = jj * (_CH * _L)
                c1s, s1s, a, b = [], [], [], []
                for t in range(_CH):
                    off = jb + t * _L
                    c1s.append(c1_v[pl.ds(off, _L)])
                    s1s.append(s1_v[pl.ds(off, _L)])
                    if first:
                        x = inv_v[pl.ds(off, _L)] * posf
                        s, c = _sincos(x)
                        s = s * jnp.float32(_SCALE)
                        c = c * jnp.float32(_SCALE)
                        a.append(c + (s - c) * m)
                        b.append(-s + (c + s) * m)
                    else:
                        a.append(av[pl.ds(off, _L)])
                        b.append(bv[pl.ds(off, _L)])
                for p in range(_GROUP):
                    for t in range(_CH):
                        buf[p, pl.ds(jb + t * _L, _L)] = a[t]
                        a[t], b[t] = (a[t] * c1s[t] + b[t] * s1s[t],
                                      b[t] * c1s[t] - a[t] * s1s[t])
                for t in range(_CH):
                    off = jb + t * _L
                    av[pl.ds(off, _L)] = a[t]
                    bv[pl.ds(off, _L)] = b[t]
                return 0

            lax.fori_loop(0, nchunk // _CH, chunk_body, 0)

            if g == 0:
                @pl.when(wid == 0)
                def _zero_row():
                    for j in range(nchunk):
                        buf[0, pl.ds(j * _L, _L)] = jnp.zeros((_L,), jnp.float32)

            handles = [
                pltpu.async_copy(
                    buf, out_hbm.at[n, pl.ds(base + g * _GROUP, _GROUP)],
                    sem)
                for n in range(N)
            ]
            pending.append(handles)

        for hs in pending:
            for h in hs:
                h.wait()

    return posenc


def kernel(inputs):
    N, T = inputs.shape[0], inputs.shape[1]
    inv, c1, s1 = _consts()
    return _make_posenc(N, T)(inv, c1, s1)
